# baseline (speedup 1.0000x reference)
import functools

import jax
import jax.numpy as jnp
from jax import lax
from jax.experimental import pallas as pl
from jax.experimental.pallas import tpu as pltpu
from jax.experimental.pallas import tpu_sc as plsc

B = 16384
K = 32
NC = 2
NS = 16
NW = NC * NS
BPW = B // NW
L = 16
GROUPS = BPW // L


def _sc_dot(user_ids, item_ids, user_table, item_table):
    mesh = plsc.VectorSubcoreMesh(core_axis_name="c", subcore_axis_name="s")

    @functools.partial(
        pl.kernel,
        mesh=mesh,
        out_type=jax.ShapeDtypeStruct((B,), jnp.float32),
        compiler_params=pltpu.CompilerParams(use_tc_tiling_on_sc=False),
        scratch_types=[
            pltpu.VMEM((BPW,), jnp.int32),
            pltpu.VMEM((BPW,), jnp.int32),
            pltpu.VMEM((BPW, K), jnp.float32),
            pltpu.VMEM((BPW, K), jnp.float32),
            pltpu.VMEM((BPW,), jnp.float32),
            pltpu.SemaphoreType.DMA,
            pltpu.SemaphoreType.DMA,
        ],
    )
    def k(uid_hbm, iid_hbm, ut_hbm, it_hbm, out_hbm,
          uidx_v, iidx_v, urows_v, irows_v, out_v, sem_u, sem_i):
        wid = lax.axis_index("s") * NC + lax.axis_index("c")
        base = wid * BPW
        pltpu.sync_copy(uid_hbm.at[pl.ds(base, BPW)], uidx_v)
        pltpu.sync_copy(iid_hbm.at[pl.ds(base, BPW)], iidx_v)
        cu = pltpu.async_copy(ut_hbm.at[uidx_v], urows_v, sem_u)
        ci = pltpu.async_copy(it_hbm.at[iidx_v], irows_v, sem_i)
        cu.wait()
        ci.wait()

        lanes = lax.iota(jnp.int32, L)
        perms = [(lanes + (8 >> s)) & 15 for s in range(4)]

        def body(g, carry):
            rbase = g * L
            acc = jnp.zeros((L,), jnp.float32)
            for j in range(L):
                r = rbase + j
                u0 = urows_v[r, pl.ds(0, L)]
                u1 = urows_v[r, pl.ds(L, L)]
                v0 = irows_v[r, pl.ds(0, L)]
                v1 = irows_v[r, pl.ds(L, L)]
                p = u0 * v0 + u1 * v1
                for s in range(4):
                    p = p + jnp.take(p, perms[s], axis=0)
                acc = jnp.where(lanes == j, p, acc)
            out_v[pl.ds(rbase, L)] = acc
            return carry

        lax.fori_loop(0, GROUPS, body, 0)
        pltpu.sync_copy(out_v, out_hbm.at[pl.ds(base, BPW)])

    return k(user_ids, item_ids, user_table, item_table)


def kernel(user_ids, item_ids, user_table, item_table):
    out = _sc_dot(user_ids[:, 0], item_ids[:, 0], user_table, item_table)
    return out[:, None]
